# baseline (speedup 1.0000x reference)
import functools

import jax
import jax.numpy as jnp
from jax import lax
from jax.experimental import pallas as pl
from jax.experimental.pallas import tpu as pltpu
from jax.experimental.pallas import tpu_sc as plsc

N_NODES = 10000
N_EDGES = 320000
D = 128
LANES = 16

NC = 2
NS = 16
NW = NC * NS
BLK = 80
NBT = N_EDGES // BLK
BPW = NBT // NW
N_PAD = 10240
ROWS_T = N_PAD // NS


def _sc_cfconv(x, Wij, ii_blocks, ij_blocks):
    mesh = plsc.VectorSubcoreMesh(core_axis_name="c", subcore_axis_name="s")

    @functools.partial(
        pl.kernel,
        out_type=jax.ShapeDtypeStruct((NC, N_PAD, D), jnp.float32),
        mesh=mesh,
        compiler_params=pltpu.CompilerParams(use_tc_tiling_on_sc=False),
        scratch_types=[
            [pltpu.VMEM((1, BLK), jnp.int32)] * 2,
            [pltpu.VMEM((1, BLK), jnp.int32)] * 2,
            [pltpu.VMEM((1, BLK), jnp.int32)] * 2,
            [pltpu.VMEM((BLK, D), jnp.float32)] * 2,
            [pltpu.VMEM((BLK, D), jnp.float32)] * 2,
            pltpu.VMEM_SHARED((N_PAD, D), jnp.float32),
            [pltpu.SemaphoreType.DMA] * 2,
            [pltpu.SemaphoreType.DMA] * 2,
            [pltpu.SemaphoreType.DMA] * 2,
        ],
    )
    def k(x_hbm, w_hbm, ii_hbm, ij_hbm, out_hbm, ii_v, iis_v, ij_v, xr_v,
          w_v, acc_sh, dsem, isem, ssem):
        c = lax.axis_index("c")
        s = lax.axis_index("s")
        w = c * NS + s
        start = w * BPW

        def zrow(r, carry):
            for p in range(D // LANES):
                xr_v[0][r, pl.ds(p * LANES, LANES)] = jnp.zeros(
                    (LANES,), jnp.float32)
                xr_v[1][r, pl.ds(p * LANES, LANES)] = jnp.zeros(
                    (LANES,), jnp.float32)
            return carry
        lax.fori_loop(0, BLK, zrow, 0)
        for q in range(2):
            for p in range(BLK // LANES):
                iis_v[q][0, pl.ds(p * LANES, LANES)] = jnp.zeros(
                    (LANES,), jnp.int32)

        base_r = s * ROWS_T
        for j in range(ROWS_T // BLK):
            pltpu.sync_copy(xr_v[0], acc_sh.at[pl.ds(base_r + j * BLK, BLK)])
        plsc.subcore_barrier()

        def load_idx(t, q):
            g = start + jnp.minimum(t, BPW - 1)
            pltpu.async_copy(ii_hbm.at[g], ii_v[q], isem[q])
            pltpu.async_copy(ij_hbm.at[g], ij_v[q], isem[q])

        def wait_idx(q):
            pltpu.make_async_copy(ii_hbm.at[0], ii_v[q], isem[q]).wait()
            pltpu.make_async_copy(ij_hbm.at[0], ij_v[q], isem[q]).wait()

        def load_data(t, q):
            g = start + t
            pltpu.async_copy(x_hbm.at[ij_v[q].at[0]], xr_v[q], dsem[q])
            pltpu.async_copy(w_hbm.at[g], w_v[q], dsem[q])

        def wait_data(q):
            pltpu.make_async_copy(x_hbm.at[pl.ds(0, BLK)], xr_v[q],
                                  dsem[q]).wait()
            pltpu.make_async_copy(w_hbm.at[0], w_v[q], dsem[q]).wait()

        def copy_sidx(q):
            for p in range(BLK // LANES):
                sl = pl.ds(p * LANES, LANES)
                iis_v[q][0, sl] = ii_v[q][0, sl]

        def scatter(q):
            pltpu.async_copy(xr_v[q], acc_sh.at[iis_v[q].at[0]], ssem[q],
                             add=True)

        def wait_scatter(q):
            pltpu.make_async_copy(xr_v[q], acc_sh.at[iis_v[q].at[0]],
                                  ssem[q]).wait()

        def compute(q):
            xr, wv = xr_v[q], w_v[q]

            def body(r, rc):
                for p in range(D // LANES):
                    sl = pl.ds(p * LANES, LANES)
                    xr[r, sl] = xr[r, sl] * wv[r, sl]
                return rc
            lax.fori_loop(0, BLK, body, 0)

        scatter(0)
        scatter(1)
        load_idx(0, 0)
        load_idx(1, 1)
        wait_idx(0)
        wait_scatter(0)
        load_data(0, 0)

        def pair(kk, carry):
            a = 2 * kk
            b = 2 * kk + 1
            wait_idx(1)
            wait_scatter(1)
            load_data(a + 1, 1)
            wait_data(0)
            copy_sidx(0)
            compute(0)
            scatter(0)
            load_idx(a + 2, 0)
            wait_idx(0)
            wait_scatter(0)
            load_data(b + 1, 0)
            wait_data(1)
            copy_sidx(1)
            compute(1)
            scatter(1)
            load_idx(b + 2, 1)
            return carry
        lax.fori_loop(0, (BPW - 1) // 2, pair, 0)

        wait_idx(1)
        wait_scatter(1)
        wait_data(0)
        copy_sidx(0)
        compute(0)
        scatter(0)
        wait_scatter(0)

        plsc.subcore_barrier()
        pltpu.sync_copy(acc_sh.at[pl.ds(base_r, ROWS_T)],
                        out_hbm.at[c].at[pl.ds(base_r, ROWS_T)])

    return k(x, Wij, ii_blocks, ij_blocks)


def _combine(yp):
    def body(a_ref, b_ref, o_ref):
        o_ref[...] = a_ref[...] + b_ref[...]

    blk = N_NODES // 10
    return pl.pallas_call(
        body,
        out_shape=jax.ShapeDtypeStruct((N_NODES, D), jnp.float32),
        grid=(10,),
        in_specs=[
            pl.BlockSpec((blk, D), lambda i: (i, 0)),
            pl.BlockSpec((blk, D), lambda i: (i, 0)),
        ],
        out_specs=pl.BlockSpec((blk, D), lambda i: (i, 0)),
    )(yp[0], yp[1])


def kernel(x, Wij, idx_i, idx_j):
    ii_blocks = idx_i.reshape(NBT, 1, BLK)
    ij_blocks = idx_j.reshape(NBT, 1, BLK)
    Wij_blocks = Wij.reshape(NBT, BLK, D)
    yp = _sc_cfconv(x, Wij_blocks, ii_blocks, ij_blocks)
    return _combine(yp)


# speedup vs baseline: 1.1455x; 1.1455x over previous
import functools

import jax
import jax.numpy as jnp
from jax import lax
from jax.experimental import pallas as pl
from jax.experimental.pallas import tpu as pltpu
from jax.experimental.pallas import tpu_sc as plsc

N_NODES = 10000
N_EDGES = 320000
D = 128
LANES = 16

NC = 2
NS = 16
NW = NC * NS
BLK = 80
NBT = N_EDGES // BLK
BPW = NBT // NW
N_PAD = 10240
ROWS_T = N_PAD // NS


def _sc_cfconv(x, Wij, ii_blocks, ij_blocks):
    mesh = plsc.VectorSubcoreMesh(core_axis_name="c", subcore_axis_name="s")

    @functools.partial(
        pl.kernel,
        out_type=jax.ShapeDtypeStruct((NC, N_PAD, D), jnp.float32),
        mesh=mesh,
        scratch_types=[
            [pltpu.VMEM((1, BLK), jnp.int32)] * 2,
            [pltpu.VMEM((1, BLK), jnp.int32)] * 2,
            [pltpu.VMEM((1, BLK), jnp.int32)] * 2,
            [pltpu.VMEM((BLK, D), jnp.float32)] * 2,
            [pltpu.VMEM((BLK, D), jnp.float32)] * 2,
            pltpu.VMEM_SHARED((N_PAD, D), jnp.float32),
            [pltpu.SemaphoreType.DMA] * 2,
            [pltpu.SemaphoreType.DMA] * 2,
            [pltpu.SemaphoreType.DMA] * 2,
        ],
    )
    def k(x_hbm, w_hbm, ii_hbm, ij_hbm, out_hbm, ii_v, iis_v, ij_v, xr_v,
          w_v, acc_sh, dsem, isem, ssem):
        c = lax.axis_index("c")
        s = lax.axis_index("s")
        w = c * NS + s
        start = w * BPW

        def zrow(r, carry):
            for p in range(D // LANES):
                xr_v[0][r, pl.ds(p * LANES, LANES)] = jnp.zeros(
                    (LANES,), jnp.float32)
                xr_v[1][r, pl.ds(p * LANES, LANES)] = jnp.zeros(
                    (LANES,), jnp.float32)
            return carry
        lax.fori_loop(0, BLK, zrow, 0)
        for q in range(2):
            for p in range(BLK // LANES):
                iis_v[q][0, pl.ds(p * LANES, LANES)] = jnp.zeros(
                    (LANES,), jnp.int32)

        base_r = s * ROWS_T
        for j in range(ROWS_T // BLK):
            pltpu.sync_copy(xr_v[0], acc_sh.at[pl.ds(base_r + j * BLK, BLK)])
        plsc.subcore_barrier()

        def load_idx(t, q):
            g = start + jnp.minimum(t, BPW - 1)
            pltpu.async_copy(ii_hbm.at[g], ii_v[q], isem[q])
            pltpu.async_copy(ij_hbm.at[g], ij_v[q], isem[q])

        def wait_idx(q):
            pltpu.make_async_copy(ii_hbm.at[0], ii_v[q], isem[q]).wait()
            pltpu.make_async_copy(ij_hbm.at[0], ij_v[q], isem[q]).wait()

        def load_data(t, q):
            g = start + t
            pltpu.async_copy(x_hbm.at[ij_v[q].at[0]], xr_v[q], dsem[q])
            pltpu.async_copy(w_hbm.at[g], w_v[q], dsem[q])

        def wait_data(q):
            pltpu.make_async_copy(x_hbm.at[pl.ds(0, BLK)], xr_v[q],
                                  dsem[q]).wait()
            pltpu.make_async_copy(w_hbm.at[0], w_v[q], dsem[q]).wait()

        def copy_sidx(q):
            for p in range(BLK // LANES):
                sl = pl.ds(p * LANES, LANES)
                iis_v[q][0, sl] = ii_v[q][0, sl]

        def scatter(q):
            pltpu.async_copy(xr_v[q], acc_sh.at[iis_v[q].at[0]], ssem[q],
                             add=True)

        def wait_scatter(q):
            pltpu.make_async_copy(xr_v[q], acc_sh.at[iis_v[q].at[0]],
                                  ssem[q]).wait()

        def compute(q):
            xr, wv = xr_v[q], w_v[q]

            def body(r, rc):
                for p in range(D // LANES):
                    sl = pl.ds(p * LANES, LANES)
                    xr[r, sl] = xr[r, sl] * wv[r, sl]
                return rc
            lax.fori_loop(0, BLK, body, 0)

        scatter(0)
        scatter(1)
        load_idx(0, 0)
        load_idx(1, 1)
        wait_idx(0)
        wait_scatter(0)
        load_data(0, 0)

        def pair(kk, carry):
            a = 2 * kk
            b = 2 * kk + 1
            wait_idx(1)
            wait_scatter(1)
            load_data(a + 1, 1)
            wait_data(0)
            copy_sidx(0)
            compute(0)
            scatter(0)
            load_idx(a + 2, 0)
            wait_idx(0)
            wait_scatter(0)
            load_data(b + 1, 0)
            wait_data(1)
            copy_sidx(1)
            compute(1)
            scatter(1)
            load_idx(b + 2, 1)
            return carry
        lax.fori_loop(0, (BPW - 1) // 2, pair, 0)

        wait_idx(1)
        wait_scatter(1)
        wait_data(0)
        copy_sidx(0)
        compute(0)
        scatter(0)
        wait_scatter(0)

        plsc.subcore_barrier()
        pltpu.sync_copy(acc_sh.at[pl.ds(base_r, ROWS_T)],
                        out_hbm.at[c].at[pl.ds(base_r, ROWS_T)])

    return k(x, Wij, ii_blocks, ij_blocks)


def _combine(yp):
    def body(a_ref, b_ref, o_ref):
        o_ref[...] = a_ref[...] + b_ref[...]

    blk = N_NODES // 10
    return pl.pallas_call(
        body,
        out_shape=jax.ShapeDtypeStruct((N_NODES, D), jnp.float32),
        grid=(10,),
        in_specs=[
            pl.BlockSpec((blk, D), lambda i: (i, 0)),
            pl.BlockSpec((blk, D), lambda i: (i, 0)),
        ],
        out_specs=pl.BlockSpec((blk, D), lambda i: (i, 0)),
    )(yp[0], yp[1])


def kernel(x, Wij, idx_i, idx_j):
    ii_blocks = idx_i.reshape(NBT, 1, BLK)
    ij_blocks = idx_j.reshape(NBT, 1, BLK)
    Wij_blocks = Wij.reshape(NBT, BLK, D)
    yp = _sc_cfconv(x, Wij_blocks, ii_blocks, ij_blocks)
    return _combine(yp)


# speedup vs baseline: 1.1558x; 1.0090x over previous
import functools

import jax
import jax.numpy as jnp
from jax import lax
from jax.experimental import pallas as pl
from jax.experimental.pallas import tpu as pltpu
from jax.experimental.pallas import tpu_sc as plsc

N_NODES = 10000
N_EDGES = 320000
D = 128
LANES = 16

NC = 2
NS = 16
NW = NC * NS
BLK = 80
NBT = N_EDGES // BLK
BPW = NBT // NW
N_PAD = 10240
ROWS_T = N_PAD // NS


def _sc_cfconv(x, Wij, ic_blocks):
    mesh = plsc.VectorSubcoreMesh(core_axis_name="c", subcore_axis_name="s")

    @functools.partial(
        pl.kernel,
        out_type=jax.ShapeDtypeStruct((NC, N_PAD, D), jnp.float32),
        mesh=mesh,
        scratch_types=[
            [pltpu.VMEM((2, BLK), jnp.int32)] * 2,
            [pltpu.VMEM((1, BLK), jnp.int32)] * 2,
            [pltpu.VMEM((BLK, D), jnp.float32)] * 2,
            [pltpu.VMEM((BLK, D), jnp.float32)] * 2,
            pltpu.VMEM_SHARED((N_PAD, D), jnp.float32),
            [pltpu.SemaphoreType.DMA] * 2,
            [pltpu.SemaphoreType.DMA] * 2,
            [pltpu.SemaphoreType.DMA] * 2,
        ],
    )
    def k(x_hbm, w_hbm, ic_hbm, out_hbm, ic_v, iis_v, xr_v,
          w_v, acc_sh, dsem, isem, ssem):
        c = lax.axis_index("c")
        s = lax.axis_index("s")
        w = c * NS + s
        start = w * BPW

        def zrow(r, carry):
            for p in range(D // LANES):
                xr_v[0][r, pl.ds(p * LANES, LANES)] = jnp.zeros(
                    (LANES,), jnp.float32)
                xr_v[1][r, pl.ds(p * LANES, LANES)] = jnp.zeros(
                    (LANES,), jnp.float32)
            return carry
        lax.fori_loop(0, BLK, zrow, 0)
        for q in range(2):
            for p in range(BLK // LANES):
                iis_v[q][0, pl.ds(p * LANES, LANES)] = jnp.zeros(
                    (LANES,), jnp.int32)

        base_r = s * ROWS_T
        for j in range(ROWS_T // BLK):
            pltpu.sync_copy(xr_v[0], acc_sh.at[pl.ds(base_r + j * BLK, BLK)])
        plsc.subcore_barrier()

        def load_idx(t, q):
            g = start + jnp.minimum(t, BPW - 1)
            pltpu.async_copy(ic_hbm.at[g], ic_v[q], isem[q])

        def wait_idx(q):
            pltpu.make_async_copy(ic_hbm.at[0], ic_v[q], isem[q]).wait()

        def load_data(t, q):
            g = start + t
            pltpu.async_copy(x_hbm.at[ic_v[q].at[1]], xr_v[q], dsem[q])
            pltpu.async_copy(w_hbm.at[g], w_v[q], dsem[q])

        def wait_data(q):
            pltpu.make_async_copy(x_hbm.at[pl.ds(0, BLK)], xr_v[q],
                                  dsem[q]).wait()
            pltpu.make_async_copy(w_hbm.at[0], w_v[q], dsem[q]).wait()

        def copy_sidx(q):
            for p in range(BLK // LANES):
                sl = pl.ds(p * LANES, LANES)
                iis_v[q][0, sl] = ic_v[q][0, sl]

        def scatter(q):
            pltpu.async_copy(xr_v[q], acc_sh.at[iis_v[q].at[0]], ssem[q],
                             add=True)

        def wait_scatter(q):
            pltpu.make_async_copy(xr_v[q], acc_sh.at[iis_v[q].at[0]],
                                  ssem[q]).wait()

        def compute(q):
            xr, wv = xr_v[q], w_v[q]

            def body(h, rc):
                r = pl.multiple_of(2 * h, 2)
                for rr in (r, r + 1):
                    for p in range(D // LANES):
                        sl = pl.ds(p * LANES, LANES)
                        xr[rr, sl] = xr[rr, sl] * wv[rr, sl]
                return rc
            lax.fori_loop(0, BLK // 2, body, 0)

        scatter(0)
        scatter(1)
        load_idx(0, 0)
        load_idx(1, 1)
        wait_idx(0)
        wait_scatter(0)
        load_data(0, 0)

        def pair(kk, carry):
            a = 2 * kk
            b = 2 * kk + 1
            wait_idx(1)
            wait_scatter(1)
            load_data(a + 1, 1)
            wait_data(0)
            copy_sidx(0)
            compute(0)
            scatter(0)
            load_idx(a + 2, 0)
            wait_idx(0)
            wait_scatter(0)
            load_data(b + 1, 0)
            wait_data(1)
            copy_sidx(1)
            compute(1)
            scatter(1)
            load_idx(b + 2, 1)
            return carry
        lax.fori_loop(0, (BPW - 1) // 2, pair, 0)

        wait_idx(1)
        wait_scatter(1)
        wait_data(0)
        copy_sidx(0)
        compute(0)
        scatter(0)
        wait_scatter(0)

        plsc.subcore_barrier()
        pltpu.sync_copy(acc_sh.at[pl.ds(base_r, ROWS_T)],
                        out_hbm.at[c].at[pl.ds(base_r, ROWS_T)])

    return k(x, Wij, ic_blocks)


def _combine(yp):
    def body(a_ref, b_ref, o_ref):
        o_ref[...] = a_ref[...] + b_ref[...]

    blk = N_NODES // 10
    return pl.pallas_call(
        body,
        out_shape=jax.ShapeDtypeStruct((N_NODES, D), jnp.float32),
        grid=(10,),
        in_specs=[
            pl.BlockSpec((blk, D), lambda i: (i, 0)),
            pl.BlockSpec((blk, D), lambda i: (i, 0)),
        ],
        out_specs=pl.BlockSpec((blk, D), lambda i: (i, 0)),
    )(yp[0], yp[1])


def kernel(x, Wij, idx_i, idx_j):
    ic_blocks = jnp.stack(
        [idx_i.reshape(NBT, BLK), idx_j.reshape(NBT, BLK)], axis=1)
    Wij_blocks = Wij.reshape(NBT, BLK, D)
    yp = _sc_cfconv(x, Wij_blocks, ic_blocks)
    return _combine(yp)
